# Initial kernel scaffold; baseline (speedup 1.0000x reference)
import jax
import jax.numpy as jnp
from jax import lax
from jax.experimental import pallas as pl
from jax.experimental.pallas import tpu as pltpu
from jax.experimental.pallas import tpu_sc as plsc

N = 10000
E = 320000
D = 128
H = 128
OUT = 128

NC = 2
NS = 16
NW = NC * NS
EPW = E // NW
CH = 80
NCHUNK = EPW // CH
ROWS_PT = N // NS

_mesh = plsc.VectorSubcoreMesh(
    core_axis_name="c", subcore_axis_name="s", num_cores=NC, num_subcores=NS)


def _deg_body(dst_hbm, w_hbm, out_hbm, dst_v, w_v, zb_v, deg_sh):
  cid = lax.axis_index("c")
  sid = lax.axis_index("s")
  wid = cid * NS + sid

  @pl.when(sid == 0)
  def _init():
    def z(i, carry):
      zb_v[pl.ds(i * 16, 16)] = jnp.zeros((16,), jnp.float32)
      return carry
    lax.fori_loop(0, N // 16, z, None)
    pltpu.sync_copy(zb_v, deg_sh)

  plsc.subcore_barrier()

  pltpu.sync_copy(dst_hbm.at[pl.ds(wid * NCHUNK, NCHUNK)], dst_v)
  pltpu.sync_copy(w_hbm.at[pl.ds(wid * NCHUNK, NCHUNK)], w_v)

  def chunk(c, carry):
    pltpu.sync_copy(w_v.at[c], deg_sh.at[dst_v.at[c]], add=True)
    return carry
  lax.fori_loop(0, NCHUNK, chunk, None)

  plsc.subcore_barrier()

  @pl.when(sid < 10)
  def _out():
    pltpu.sync_copy(deg_sh.at[pl.ds(sid * 1000, 1000)],
                    out_hbm.at[cid, pl.ds(sid * 1000, 1000)])


_sc_deg = pl.kernel(
    _deg_body,
    out_type=jax.ShapeDtypeStruct((NC, N), jnp.float32),
    mesh=_mesh,
    scratch_types=[
        pltpu.VMEM((NCHUNK, CH), jnp.int32),
        pltpu.VMEM((NCHUNK, CH), jnp.float32),
        pltpu.VMEM((N,), jnp.float32),
        pltpu.VMEM_SHARED((N,), jnp.float32),
    ],
)


def _agg_body(g_hbm, src_hbm, dst_hbm, w_hbm, out_hbm,
              src_v, dst_v, w_v, rows_v, zb_v, sem, acc_sh):
  cid = lax.axis_index("c")
  sid = lax.axis_index("s")
  wid = cid * NS + sid

  def z(i, carry):
    def zj(j, c2):
      zb_v[i, pl.ds(j * 16, 16)] = jnp.zeros((16,), jnp.float32)
      return c2
    lax.fori_loop(0, H // 16, zj, None)
    return carry
  lax.fori_loop(0, 125, z, None)
  for k in range(ROWS_PT // 125):
    pltpu.sync_copy(zb_v, acc_sh.at[pl.ds(sid * ROWS_PT + k * 125, 125)])

  plsc.subcore_barrier()

  pltpu.sync_copy(src_hbm.at[pl.ds(wid * NCHUNK, NCHUNK)], src_v)
  pltpu.sync_copy(dst_hbm.at[pl.ds(wid * NCHUNK, NCHUNK)], dst_v)
  pltpu.sync_copy(w_hbm.at[pl.ds(wid * NCHUNK, NCHUNK)], w_v)

  def chunk(c, carry):
    pltpu.async_copy(g_hbm.at[src_v.at[c]], rows_v, sem).wait()
    def scale(e, c2):
      wsplat = plsc.load_gather(
          w_v, [jnp.full((16,), c, jnp.int32), jnp.full((16,), e, jnp.int32)])
      for j in range(H // 16):
        sl = pl.ds(j * 16, 16)
        rows_v[e, sl] = rows_v[e, sl] * wsplat
      return c2
    lax.fori_loop(0, CH, scale, None)
    pltpu.sync_copy(rows_v, acc_sh.at[dst_v.at[c]], add=True)
    return carry
  lax.fori_loop(0, NCHUNK, chunk, None)

  plsc.subcore_barrier()

  pltpu.sync_copy(acc_sh.at[pl.ds(sid * ROWS_PT, ROWS_PT)],
                  out_hbm.at[cid, pl.ds(sid * ROWS_PT, ROWS_PT)])


_sc_agg = pl.kernel(
    _agg_body,
    out_type=jax.ShapeDtypeStruct((NC, N, H), jnp.float32),
    mesh=_mesh,
    scratch_types=[
        pltpu.VMEM((NCHUNK, CH), jnp.int32),
        pltpu.VMEM((NCHUNK, CH), jnp.int32),
        pltpu.VMEM((NCHUNK, CH), jnp.float32),
        pltpu.VMEM((CH, H), jnp.float32),
        pltpu.VMEM((125, H), jnp.float32),
        pltpu.SemaphoreType.DMA,
        pltpu.VMEM_SHARED((N, H), jnp.float32),
    ],
)


BR = 500


def _tc1_body(degp_ref, x_ref, w1_ref, dinv_ref, g1_ref):
  deg = degp_ref[0] + degp_ref[1] + 1.0
  dinv = lax.rsqrt(deg)
  h = jnp.dot(x_ref[...], w1_ref[...], preferred_element_type=jnp.float32)
  dinv_ref[...] = dinv
  g1_ref[...] = dinv * h


def _tc_mid_body(acc_ref, g_ref, dinv_ref, b_ref, w_ref, gnext_ref):
  a = acc_ref[0] + acc_ref[1] + g_ref[...]
  o = jnp.maximum(dinv_ref[...] * a + b_ref[...], 0.0)
  gnext_ref[...] = dinv_ref[...] * jnp.dot(
      o, w_ref[...], preferred_element_type=jnp.float32)


def _tc_out_body(acc_ref, g_ref, dinv_ref, b_ref, w_ref, bfc_ref, y_ref):
  a = acc_ref[0] + acc_ref[1] + g_ref[...]
  o = jnp.maximum(dinv_ref[...] * a + b_ref[...], 0.0)
  y_ref[...] = jnp.dot(
      o, w_ref[...], preferred_element_type=jnp.float32) + bfc_ref[...]


_tc1 = pl.pallas_call(
    _tc1_body,
    grid=(N // BR,),
    in_specs=[
        pl.BlockSpec((NC, BR, 1), lambda i: (0, i, 0)),
        pl.BlockSpec((BR, D), lambda i: (i, 0)),
        pl.BlockSpec((D, H), lambda i: (0, 0)),
    ],
    out_specs=[
        pl.BlockSpec((BR, 1), lambda i: (i, 0)),
        pl.BlockSpec((BR, H), lambda i: (i, 0)),
    ],
    out_shape=[
        jax.ShapeDtypeStruct((N, 1), jnp.float32),
        jax.ShapeDtypeStruct((N, H), jnp.float32),
    ],
)

_tc_mid = pl.pallas_call(
    _tc_mid_body,
    grid=(N // BR,),
    in_specs=[
        pl.BlockSpec((NC, BR, H), lambda i: (0, i, 0)),
        pl.BlockSpec((BR, H), lambda i: (i, 0)),
        pl.BlockSpec((BR, 1), lambda i: (i, 0)),
        pl.BlockSpec((1, H), lambda i: (0, 0)),
        pl.BlockSpec((H, H), lambda i: (0, 0)),
    ],
    out_specs=pl.BlockSpec((BR, H), lambda i: (i, 0)),
    out_shape=jax.ShapeDtypeStruct((N, H), jnp.float32),
)

_tc_out = pl.pallas_call(
    _tc_out_body,
    grid=(N // BR,),
    in_specs=[
        pl.BlockSpec((NC, BR, H), lambda i: (0, i, 0)),
        pl.BlockSpec((BR, H), lambda i: (i, 0)),
        pl.BlockSpec((BR, 1), lambda i: (i, 0)),
        pl.BlockSpec((1, H), lambda i: (0, 0)),
        pl.BlockSpec((H, OUT), lambda i: (0, 0)),
        pl.BlockSpec((1, OUT), lambda i: (0, 0)),
    ],
    out_specs=pl.BlockSpec((BR, OUT), lambda i: (i, 0)),
    out_shape=jax.ShapeDtypeStruct((N, OUT), jnp.float32),
)


def kernel(x, edge_index, edge_weight, W1, b1, W2, b2, Wfc, bfc):
  src = edge_index[0].reshape(NW * NCHUNK, CH)
  dst = edge_index[1].reshape(NW * NCHUNK, CH)
  w = edge_weight.reshape(NW * NCHUNK, CH)

  degp = _sc_deg(dst, w)
  dinv, g1 = _tc1(degp.reshape(NC, N, 1), x, W1)
  acc1 = _sc_agg(g1, src, dst, w)
  g2 = _tc_mid(acc1, g1, dinv, b1.reshape(1, H), W2)
  acc2 = _sc_agg(g2, src, dst, w)
  return _tc_out(acc2, g2, dinv, b2.reshape(1, H), Wfc, bfc.reshape(1, OUT))


# speedup vs baseline: 9.0577x; 9.0577x over previous
import jax
import jax.numpy as jnp
from jax import lax
from jax.experimental import pallas as pl
from jax.experimental.pallas import tpu as pltpu
from jax.experimental.pallas import tpu_sc as plsc

N = 10000
E = 320000
D = 128
H = 128
OUT = 128

NC = 2
NS = 16
NW = NC * NS
EPW = E // NW
CH = 80
NCHUNK = EPW // CH
ROWS_PT = N // NS

_mesh = plsc.VectorSubcoreMesh(
    core_axis_name="c", subcore_axis_name="s", num_cores=NC, num_subcores=NS)


def _deg_body(dst_hbm, w_hbm, out_hbm, dst_c, w_c, zb_v, deg_sh):
  cid = lax.axis_index("c")
  sid = lax.axis_index("s")
  wid = cid * NS + sid

  @pl.when(sid == 0)
  def _init():
    def z(i, carry):
      zb_v[pl.ds(i * 16, 16)] = jnp.zeros((16,), jnp.float32)
      return carry
    lax.fori_loop(0, N // 16, z, None)
    pltpu.sync_copy(zb_v, deg_sh)

  plsc.subcore_barrier()

  def chunk(c, carry):
    base = wid * EPW + c * CH
    pltpu.sync_copy(dst_hbm.at[pl.ds(base, CH)], dst_c)
    pltpu.sync_copy(w_hbm.at[pl.ds(base, CH)], w_c)
    pltpu.sync_copy(w_c, deg_sh.at[dst_c], add=True)
    return carry
  lax.fori_loop(0, NCHUNK, chunk, None)

  plsc.subcore_barrier()

  @pl.when(sid < 10)
  def _out():
    stage = zb_v.at[pl.ds(0, 1000)]
    pltpu.sync_copy(deg_sh.at[pl.ds(sid * 1000, 1000)], stage)
    pltpu.sync_copy(stage, out_hbm.at[pl.ds(cid * N + sid * 1000, 1000)])


_sc_deg = pl.kernel(
    _deg_body,
    out_type=jax.ShapeDtypeStruct((NC * N,), jnp.float32),
    mesh=_mesh,
    compiler_params=pltpu.CompilerParams(needs_layout_passes=False),
    scratch_types=[
        pltpu.VMEM((CH,), jnp.int32),
        pltpu.VMEM((CH,), jnp.float32),
        pltpu.VMEM((N,), jnp.float32),
        pltpu.VMEM_SHARED((N,), jnp.float32),
    ],
)


def _agg_body(g_hbm, src_hbm, dst_hbm, w_hbm, out_hbm,
              src_c, dst_c, w_c, rows_v, zb_v, sem, acc_sh):
  cid = lax.axis_index("c")
  sid = lax.axis_index("s")
  wid = cid * NS + sid

  def z(i, carry):
    def zj(j, c2):
      zb_v[i, pl.ds(j * 16, 16)] = jnp.zeros((16,), jnp.float32)
      return c2
    lax.fori_loop(0, H // 16, zj, None)
    return carry
  lax.fori_loop(0, 200, z, None)

  @pl.when(sid < 10)
  def _zero():
    for k in range(5):
      pltpu.sync_copy(zb_v, acc_sh.at[pl.ds(sid * 1000 + k * 200, 200)])

  plsc.subcore_barrier()

  def chunk(c, carry):
    base = wid * EPW + c * CH
    pltpu.sync_copy(src_hbm.at[pl.ds(base, CH)], src_c)
    pltpu.sync_copy(dst_hbm.at[pl.ds(base, CH)], dst_c)
    pltpu.sync_copy(w_hbm.at[pl.ds(base, CH)], w_c)
    pltpu.async_copy(g_hbm.at[src_c], rows_v, sem).wait()
    def scale(e, c2):
      wsplat = plsc.load_gather(w_c, [jnp.full((16,), e, jnp.int32)])
      for j in range(H // 16):
        sl = pl.ds(j * 16, 16)
        rows_v[e, sl] = rows_v[e, sl] * wsplat
      return c2
    lax.fori_loop(0, CH, scale, None)
    pltpu.sync_copy(rows_v, acc_sh.at[dst_c], add=True)
    return carry
  lax.fori_loop(0, NCHUNK, chunk, None)

  plsc.subcore_barrier()

  @pl.when(sid < 10)
  def _out():
    for k in range(5):
      pltpu.sync_copy(acc_sh.at[pl.ds(sid * 1000 + k * 200, 200)], zb_v)
      pltpu.sync_copy(zb_v, out_hbm.at[cid, pl.ds(sid * 1000 + k * 200, 200)])


_sc_agg = pl.kernel(
    _agg_body,
    out_type=jax.ShapeDtypeStruct((NC, N, H), jnp.float32),
    mesh=_mesh,
    compiler_params=pltpu.CompilerParams(needs_layout_passes=False),
    scratch_types=[
        pltpu.VMEM((CH,), jnp.int32),
        pltpu.VMEM((CH,), jnp.int32),
        pltpu.VMEM((CH,), jnp.float32),
        pltpu.VMEM((CH, H), jnp.float32),
        pltpu.VMEM((200, H), jnp.float32),
        pltpu.SemaphoreType.DMA,
        pltpu.VMEM_SHARED((N, H), jnp.float32),
    ],
)


BR = 1000


def _tc1_body(degp_ref, x_ref, w1_ref, dinv_ref, g1_ref):
  deg = degp_ref[0] + degp_ref[1] + 1.0
  dinv = lax.rsqrt(deg)
  h = jnp.dot(x_ref[...], w1_ref[...], preferred_element_type=jnp.float32)
  dinv_ref[...] = dinv
  g1_ref[...] = dinv * h


def _tc_mid_body(acc_ref, g_ref, dinv_ref, b_ref, w_ref, gnext_ref):
  a = acc_ref[0] + acc_ref[1] + g_ref[...]
  o = jnp.maximum(dinv_ref[...] * a + b_ref[...], 0.0)
  gnext_ref[...] = dinv_ref[...] * jnp.dot(
      o, w_ref[...], preferred_element_type=jnp.float32)


def _tc_out_body(acc_ref, g_ref, dinv_ref, b_ref, w_ref, bfc_ref, y_ref):
  a = acc_ref[0] + acc_ref[1] + g_ref[...]
  o = jnp.maximum(dinv_ref[...] * a + b_ref[...], 0.0)
  y_ref[...] = jnp.dot(
      o, w_ref[...], preferred_element_type=jnp.float32) + bfc_ref[...]


_tc1 = pl.pallas_call(
    _tc1_body,
    grid=(N // BR,),
    in_specs=[
        pl.BlockSpec((NC, BR, H), lambda i: (0, i, 0)),
        pl.BlockSpec((BR, D), lambda i: (i, 0)),
        pl.BlockSpec((D, H), lambda i: (0, 0)),
    ],
    out_specs=[
        pl.BlockSpec((BR, H), lambda i: (i, 0)),
        pl.BlockSpec((BR, H), lambda i: (i, 0)),
    ],
    out_shape=[
        jax.ShapeDtypeStruct((N, H), jnp.float32),
        jax.ShapeDtypeStruct((N, H), jnp.float32),
    ],
)

_tc_mid = pl.pallas_call(
    _tc_mid_body,
    grid=(N // BR,),
    in_specs=[
        pl.BlockSpec((NC, BR, H), lambda i: (0, i, 0)),
        pl.BlockSpec((BR, H), lambda i: (i, 0)),
        pl.BlockSpec((BR, H), lambda i: (i, 0)),
        pl.BlockSpec((1, H), lambda i: (0, 0)),
        pl.BlockSpec((H, H), lambda i: (0, 0)),
    ],
    out_specs=pl.BlockSpec((BR, H), lambda i: (i, 0)),
    out_shape=jax.ShapeDtypeStruct((N, H), jnp.float32),
)

_tc_out = pl.pallas_call(
    _tc_out_body,
    grid=(N // BR,),
    in_specs=[
        pl.BlockSpec((NC, BR, H), lambda i: (0, i, 0)),
        pl.BlockSpec((BR, H), lambda i: (i, 0)),
        pl.BlockSpec((BR, H), lambda i: (i, 0)),
        pl.BlockSpec((1, H), lambda i: (0, 0)),
        pl.BlockSpec((H, OUT), lambda i: (0, 0)),
        pl.BlockSpec((1, OUT), lambda i: (0, 0)),
    ],
    out_specs=pl.BlockSpec((BR, OUT), lambda i: (i, 0)),
    out_shape=jax.ShapeDtypeStruct((N, OUT), jnp.float32),
)


def kernel(x, edge_index, edge_weight, W1, b1, W2, b2, Wfc, bfc):
  src = edge_index[0]
  dst = edge_index[1]
  w = edge_weight

  degp = _sc_deg(dst, w)
  degb = jnp.broadcast_to(degp.reshape(NC, N, 1), (NC, N, H))
  dinv, g1 = _tc1(degb, x, W1)
  acc1 = _sc_agg(g1, src, dst, w)
  g2 = _tc_mid(acc1, g1, dinv, b1.reshape(1, H), W2)
  acc2 = _sc_agg(g2, src, dst, w)
  return _tc_out(acc2, g2, dinv, b2.reshape(1, H), Wfc, bfc.reshape(1, OUT))


# speedup vs baseline: 21.3872x; 2.3612x over previous
import jax
import jax.numpy as jnp
from jax import lax
from jax.experimental import pallas as pl
from jax.experimental.pallas import tpu as pltpu
from jax.experimental.pallas import tpu_sc as plsc

N = 10000
E = 320000
D = 128
H = 128
OUT = 128

NC = 2
NS = 16
NW = NC * NS
EPW = E // NW
CH = 40
NCHUNK = EPW // CH
NB = 5
NGG = NCHUNK // (2 * NB)

_mesh = plsc.VectorSubcoreMesh(
    core_axis_name="c", subcore_axis_name="s", num_cores=NC, num_subcores=NS)


def _deg_body(dst_hbm, w_hbm, out_hbm, w_all, zb_v,
              d00, d01, d02, d03, d04, d10, d11, d12, d13, d14,
              ds0, ds1, ds2, ds3, ds4, ss0, ss1, ss2, ss3, ss4, deg_sh):
  cid = lax.axis_index("c")
  sid = lax.axis_index("s")
  wid = cid * NS + sid
  dstb = ((d00, d01, d02, d03, d04), (d10, d11, d12, d13, d14))
  dsem = (ds0, ds1, ds2, ds3, ds4)
  ssem = (ss0, ss1, ss2, ss3, ss4)

  @pl.when(sid == 0)
  def _init():
    def z(i, carry):
      zb_v[pl.ds(i * 16, 16)] = jnp.zeros((16,), jnp.float32)
      return carry
    lax.fori_loop(0, N // 16, z, None)
    pltpu.sync_copy(zb_v, deg_sh)

  pltpu.sync_copy(w_hbm.at[pl.ds(wid * EPW, EPW)], w_all)
  plsc.subcore_barrier()

  for b in range(NB):
    pltpu.async_copy(dst_hbm.at[pl.ds(wid * EPW + b * CH, CH)],
                     dstb[0][b], dsem[b])

  def rounds(gg, carry):
    for p in range(2):
      for b in range(NB):
        c = (2 * gg + p) * NB + b
        base = c * CH
        pltpu.make_async_copy(
            dst_hbm.at[pl.ds(wid * EPW + base, CH)], dstb[p][b],
            dsem[b]).wait()
        def wait_prev():
          pltpu.make_async_copy(
              w_all.at[pl.ds(base, CH)], deg_sh.at[dstb[1 - p][b]],
              ssem[b]).wait()
        if p == 1:
          wait_prev()
        else:
          pl.when(gg > 0)(wait_prev)
        pltpu.async_copy(w_all.at[pl.ds(base, CH)], deg_sh.at[dstb[p][b]],
                         ssem[b], add=True)
        def prefetch():
          pltpu.async_copy(
              dst_hbm.at[pl.ds(wid * EPW + base + NB * CH, CH)],
              dstb[1 - p][b], dsem[b])
        if p == 0:
          prefetch()
        else:
          pl.when(gg < NGG - 1)(prefetch)
    return carry
  lax.fori_loop(0, NGG, rounds, None)

  for b in range(NB):
    pltpu.make_async_copy(w_all.at[pl.ds(0, CH)], deg_sh.at[dstb[1][b]],
                          ssem[b]).wait()

  plsc.subcore_barrier()

  @pl.when(sid < 10)
  def _out():
    stage = zb_v.at[pl.ds(0, 1000)]
    pltpu.sync_copy(deg_sh.at[pl.ds(sid * 1000, 1000)], stage)
    pltpu.sync_copy(stage, out_hbm.at[pl.ds(cid * N + sid * 1000, 1000)])


_sc_deg = pl.kernel(
    _deg_body,
    out_type=jax.ShapeDtypeStruct((NC * N,), jnp.float32),
    mesh=_mesh,
    compiler_params=pltpu.CompilerParams(needs_layout_passes=False),
    scratch_types=(
        [pltpu.VMEM((EPW,), jnp.float32),
         pltpu.VMEM((N,), jnp.float32)]
        + [pltpu.VMEM((CH,), jnp.int32)] * 10
        + [pltpu.SemaphoreType.DMA] * 10
        + [pltpu.VMEM_SHARED((N,), jnp.float32)]
    ),
)


def _agg_body(g_hbm, src_hbm, dst_hbm, w_hbm, out_hbm,
              src_all, w_all,
              ri0, ri1, ri2, ri3, ri4, d0, d1, d2, d3, d4,
              gs0, gs1, gs2, gs3, gs4, ss0, ss1, ss2, ss3, ss4, acc_sh):
  cid = lax.axis_index("c")
  sid = lax.axis_index("s")
  wid = cid * NS + sid
  rin = (ri0, ri1, ri2, ri3, ri4)
  dstb = (d0, d1, d2, d3, d4)
  gsem = (gs0, gs1, gs2, gs3, gs4)
  ssem = (ss0, ss1, ss2, ss3, ss4)
  NR = NCHUNK // NB

  def issue(c, b):
    pltpu.async_copy(dst_hbm.at[pl.ds(wid * EPW + c * CH, CH)],
                     dstb[b], gsem[b])
    pltpu.async_copy(g_hbm.at[src_all.at[pl.ds(c * CH, CH)]], rin[b], gsem[b])

  def z(i, carry):
    for j in range(H // 16):
      rin[0][i, pl.ds(j * 16, 16)] = jnp.zeros((16,), jnp.float32)
    return carry
  lax.fori_loop(0, CH, z, None)

  @pl.when(sid < 10)
  def _zero():
    for k in range(25):
      pltpu.sync_copy(rin[0], acc_sh.at[pl.ds(sid * 1000 + k * CH, CH)])

  pltpu.sync_copy(src_hbm.at[pl.ds(wid * EPW, EPW)], src_all)
  pltpu.sync_copy(w_hbm.at[pl.ds(wid * EPW, EPW)], w_all)

  plsc.subcore_barrier()

  issue(0, 0)

  def rounds(r, carry):
    for b in range(NB):
      c = r * NB + b
      bn = (b + 1) % NB
      def advance():
        def wait_prev():
          pltpu.make_async_copy(
              rin[bn], acc_sh.at[dstb[bn]], ssem[bn]).wait()
        if b < NB - 1:
          pl.when(r > 0)(wait_prev)
        else:
          wait_prev()
        issue(c + 1, bn)
      if b < NB - 1:
        advance()
      else:
        pl.when(r < NR - 1)(advance)

      pltpu.make_async_copy(dst_hbm.at[pl.ds(wid * EPW + c * CH, CH)],
                            dstb[b], gsem[b]).wait()
      pltpu.make_async_copy(g_hbm.at[src_all.at[pl.ds(c * CH, CH)]],
                            rin[b], gsem[b]).wait()

      def scale(e, c2):
        wsplat = plsc.load_gather(
            w_all, [jnp.full((16,), c * CH + e, jnp.int32)])
        for j in range(H // 16):
          sl = pl.ds(j * 16, 16)
          rin[b][e, sl] = rin[b][e, sl] * wsplat
        return c2
      lax.fori_loop(0, CH, scale, None)

      pltpu.async_copy(rin[b], acc_sh.at[dstb[b]], ssem[b], add=True)
    return carry
  lax.fori_loop(0, NR, rounds, None)

  for b in range(NB):
    pltpu.make_async_copy(rin[b], acc_sh.at[dstb[b]], ssem[b]).wait()

  plsc.subcore_barrier()

  @pl.when(sid < 10)
  def _out():
    for k in range(25):
      sl = pl.ds(sid * 1000 + k * CH, CH)
      pltpu.sync_copy(acc_sh.at[sl], rin[0])
      pltpu.sync_copy(rin[0], out_hbm.at[cid, sl])


_sc_agg = pl.kernel(
    _agg_body,
    out_type=jax.ShapeDtypeStruct((NC, N, H), jnp.float32),
    mesh=_mesh,
    compiler_params=pltpu.CompilerParams(needs_layout_passes=False),
    scratch_types=(
        [pltpu.VMEM((EPW,), jnp.int32),
         pltpu.VMEM((EPW,), jnp.float32)]
        + [pltpu.VMEM((CH, H), jnp.float32)] * 5
        + [pltpu.VMEM((CH,), jnp.int32)] * 5
        + [pltpu.SemaphoreType.DMA] * 10
        + [pltpu.VMEM_SHARED((N, H), jnp.float32)]
    ),
)


BR = 1000


def _tc1_body(degp_ref, x_ref, w1_ref, dinv_ref, g1_ref):
  deg = degp_ref[0] + degp_ref[1] + 1.0
  dinv = lax.rsqrt(deg)
  h = jnp.dot(x_ref[...], w1_ref[...], preferred_element_type=jnp.float32)
  dinv_ref[...] = dinv
  g1_ref[...] = dinv * h


def _tc_mid_body(acc_ref, g_ref, dinv_ref, b_ref, w_ref, gnext_ref):
  a = acc_ref[0] + acc_ref[1] + g_ref[...]
  o = jnp.maximum(dinv_ref[...] * a + b_ref[...], 0.0)
  gnext_ref[...] = dinv_ref[...] * jnp.dot(
      o, w_ref[...], preferred_element_type=jnp.float32)


def _tc_out_body(acc_ref, g_ref, dinv_ref, b_ref, w_ref, bfc_ref, y_ref):
  a = acc_ref[0] + acc_ref[1] + g_ref[...]
  o = jnp.maximum(dinv_ref[...] * a + b_ref[...], 0.0)
  y_ref[...] = jnp.dot(
      o, w_ref[...], preferred_element_type=jnp.float32) + bfc_ref[...]


_tc1 = pl.pallas_call(
    _tc1_body,
    grid=(N // BR,),
    in_specs=[
        pl.BlockSpec((NC, BR, H), lambda i: (0, i, 0)),
        pl.BlockSpec((BR, D), lambda i: (i, 0)),
        pl.BlockSpec((D, H), lambda i: (0, 0)),
    ],
    out_specs=[
        pl.BlockSpec((BR, H), lambda i: (i, 0)),
        pl.BlockSpec((BR, H), lambda i: (i, 0)),
    ],
    out_shape=[
        jax.ShapeDtypeStruct((N, H), jnp.float32),
        jax.ShapeDtypeStruct((N, H), jnp.float32),
    ],
)

_tc_mid = pl.pallas_call(
    _tc_mid_body,
    grid=(N // BR,),
    in_specs=[
        pl.BlockSpec((NC, BR, H), lambda i: (0, i, 0)),
        pl.BlockSpec((BR, H), lambda i: (i, 0)),
        pl.BlockSpec((BR, H), lambda i: (i, 0)),
        pl.BlockSpec((1, H), lambda i: (0, 0)),
        pl.BlockSpec((H, H), lambda i: (0, 0)),
    ],
    out_specs=pl.BlockSpec((BR, H), lambda i: (i, 0)),
    out_shape=jax.ShapeDtypeStruct((N, H), jnp.float32),
)

_tc_out = pl.pallas_call(
    _tc_out_body,
    grid=(N // BR,),
    in_specs=[
        pl.BlockSpec((NC, BR, H), lambda i: (0, i, 0)),
        pl.BlockSpec((BR, H), lambda i: (i, 0)),
        pl.BlockSpec((BR, H), lambda i: (i, 0)),
        pl.BlockSpec((1, H), lambda i: (0, 0)),
        pl.BlockSpec((H, OUT), lambda i: (0, 0)),
        pl.BlockSpec((1, OUT), lambda i: (0, 0)),
    ],
    out_specs=pl.BlockSpec((BR, OUT), lambda i: (i, 0)),
    out_shape=jax.ShapeDtypeStruct((N, OUT), jnp.float32),
)


def kernel(x, edge_index, edge_weight, W1, b1, W2, b2, Wfc, bfc):
  src = edge_index[0]
  dst = edge_index[1]
  w = edge_weight

  degp = _sc_deg(dst, w)
  degb = jnp.broadcast_to(degp.reshape(NC, N, 1), (NC, N, H))
  dinv, g1 = _tc1(degb, x, W1)
  acc1 = _sc_agg(g1, src, dst, w)
  g2 = _tc_mid(acc1, g1, dinv, b1.reshape(1, H), W2)
  acc2 = _sc_agg(g2, src, dst, w)
  return _tc_out(acc2, g2, dinv, b2.reshape(1, H), Wfc, bfc.reshape(1, OUT))


# speedup vs baseline: 25.9291x; 1.2124x over previous
import jax
import jax.numpy as jnp
from jax import lax
from jax.experimental import pallas as pl
from jax.experimental.pallas import tpu as pltpu
from jax.experimental.pallas import tpu_sc as plsc

N = 10000
E = 320000
D = 128
H = 128
OUT = 128

NC = 2
NS = 16
NW = NC * NS
EPW = E // NW
CH = 40
NCHUNK = EPW // CH
NB = 5
NGG = NCHUNK // (2 * NB)
CHA = 80
NCHA = EPW // CHA
NBA = 3

_mesh = plsc.VectorSubcoreMesh(
    core_axis_name="c", subcore_axis_name="s", num_cores=NC, num_subcores=NS)


def _deg_body(dst_hbm, w_hbm, out_hbm, w_all, zb_v,
              d00, d01, d02, d03, d04, d10, d11, d12, d13, d14,
              ds0, ds1, ds2, ds3, ds4, ss0, ss1, ss2, ss3, ss4, deg_sh):
  cid = lax.axis_index("c")
  sid = lax.axis_index("s")
  wid = cid * NS + sid
  dstb = ((d00, d01, d02, d03, d04), (d10, d11, d12, d13, d14))
  dsem = (ds0, ds1, ds2, ds3, ds4)
  ssem = (ss0, ss1, ss2, ss3, ss4)

  @pl.when(sid == 0)
  def _init():
    def z(i, carry):
      zb_v[pl.ds(i * 16, 16)] = jnp.zeros((16,), jnp.float32)
      return carry
    lax.fori_loop(0, N // 16, z, None)
    pltpu.sync_copy(zb_v, deg_sh)

  pltpu.sync_copy(w_hbm.at[pl.ds(wid * EPW, EPW)], w_all)
  plsc.subcore_barrier()

  for b in range(NB):
    pltpu.async_copy(dst_hbm.at[pl.ds(wid * EPW + b * CH, CH)],
                     dstb[0][b], dsem[b])

  def rounds(gg, carry):
    for p in range(2):
      for b in range(NB):
        c = (2 * gg + p) * NB + b
        base = c * CH
        pltpu.make_async_copy(
            dst_hbm.at[pl.ds(wid * EPW + base, CH)], dstb[p][b],
            dsem[b]).wait()
        def wait_prev():
          pltpu.make_async_copy(
              w_all.at[pl.ds(base, CH)], deg_sh.at[dstb[1 - p][b]],
              ssem[b]).wait()
        if p == 1:
          wait_prev()
        else:
          pl.when(gg > 0)(wait_prev)
        pltpu.async_copy(w_all.at[pl.ds(base, CH)], deg_sh.at[dstb[p][b]],
                         ssem[b], add=True)
        def prefetch():
          pltpu.async_copy(
              dst_hbm.at[pl.ds(wid * EPW + base + NB * CH, CH)],
              dstb[1 - p][b], dsem[b])
        if p == 0:
          prefetch()
        else:
          pl.when(gg < NGG - 1)(prefetch)
    return carry
  lax.fori_loop(0, NGG, rounds, None)

  for b in range(NB):
    pltpu.make_async_copy(w_all.at[pl.ds(0, CH)], deg_sh.at[dstb[1][b]],
                          ssem[b]).wait()

  plsc.subcore_barrier()

  @pl.when(sid < 10)
  def _out():
    stage = zb_v.at[pl.ds(0, 1000)]
    pltpu.sync_copy(deg_sh.at[pl.ds(sid * 1000, 1000)], stage)
    pltpu.sync_copy(stage, out_hbm.at[pl.ds(cid * N + sid * 1000, 1000)])


_sc_deg = pl.kernel(
    _deg_body,
    out_type=jax.ShapeDtypeStruct((NC * N,), jnp.float32),
    mesh=_mesh,
    compiler_params=pltpu.CompilerParams(needs_layout_passes=False),
    scratch_types=(
        [pltpu.VMEM((EPW,), jnp.float32),
         pltpu.VMEM((N,), jnp.float32)]
        + [pltpu.VMEM((CH,), jnp.int32)] * 10
        + [pltpu.SemaphoreType.DMA] * 10
        + [pltpu.VMEM_SHARED((N,), jnp.float32)]
    ),
)


def _agg_body(g_hbm, src_hbm, dst_hbm, w_hbm, out_hbm,
              w_all,
              ri0, ri1, ri2, d0, d1, d2, s0, s1, s2,
              gs0, gs1, gs2, ss0, ss1, ss2, ps0, ps1, ps2, acc_sh):
  cid = lax.axis_index("c")
  sid = lax.axis_index("s")
  wid = cid * NS + sid
  rin = (ri0, ri1, ri2)
  dstb = (d0, d1, d2)
  srcb = (s0, s1, s2)
  gsem = (gs0, gs1, gs2)
  ssem = (ss0, ss1, ss2)
  psem = (ps0, ps1, ps2)
  NR = 41

  def prefetch_sw(c, b):
    pltpu.async_copy(src_hbm.at[pl.ds(wid * EPW + c * CHA, CHA)],
                     srcb[b], psem[b])

  def issue(c, b):
    pltpu.make_async_copy(src_hbm.at[pl.ds(wid * EPW + c * CHA, CHA)],
                          srcb[b], psem[b]).wait()
    pltpu.async_copy(dst_hbm.at[pl.ds(wid * EPW + c * CHA, CHA)],
                     dstb[b], gsem[b])
    pltpu.async_copy(g_hbm.at[srcb[b]], rin[b], gsem[b])

  def wait_issue(c, b):
    pltpu.make_async_copy(dst_hbm.at[pl.ds(wid * EPW + c * CHA, CHA)],
                          dstb[b], gsem[b]).wait()
    pltpu.make_async_copy(g_hbm.at[srcb[b]], rin[b], gsem[b]).wait()

  def wait_scatter(b):
    pltpu.make_async_copy(rin[b], acc_sh.at[dstb[b]], ssem[b]).wait()

  def scale_chunk(c, b):
    def scale(i, c2):
      for u in range(2):
        e = 2 * i + u
        wsplat = plsc.load_gather(
            w_all, [jnp.full((16,), c * CHA + e, jnp.int32)])
        for j in range(H // 16):
          sl = pl.ds(j * 16, 16)
          rin[b][e, sl] = rin[b][e, sl] * wsplat
      return c2
    lax.fori_loop(0, CHA // 2, scale, None)

  def start_scatter(b):
    pltpu.async_copy(rin[b], acc_sh.at[dstb[b]], ssem[b], add=True)

  def z(i, carry):
    for j in range(H // 16):
      rin[0][i, pl.ds(j * 16, 16)] = jnp.zeros((16,), jnp.float32)
    return carry
  lax.fori_loop(0, CHA, z, None)

  @pl.when(sid < 10)
  def _zero():
    for k in range(12):
      pltpu.sync_copy(rin[0], acc_sh.at[pl.ds(sid * 1000 + k * CHA, CHA)])
    pltpu.sync_copy(rin[0].at[pl.ds(0, 40)],
                    acc_sh.at[pl.ds(sid * 1000 + 960, 40)])

  pltpu.sync_copy(w_hbm.at[pl.ds(wid * EPW, EPW)], w_all)

  plsc.subcore_barrier()

  prefetch_sw(0, 0)
  prefetch_sw(1, 1)
  issue(0, 0)

  def rounds(r, carry):
    for b in range(NBA):
      c = r * NBA + b
      bn = (b + 1) % NBA
      b2 = (b + 2) % NBA
      def advance():
        pl.when(r > 0)(lambda: wait_scatter(bn)) if b < NBA - 1 \
            else wait_scatter(bn)
        issue(c + 1, bn)
      advance()
      prefetch_sw(c + 2, b2)
      wait_issue(c, b)
      scale_chunk(c, b)
      start_scatter(b)
    return carry
  lax.fori_loop(0, NR, rounds, None)

  wait_scatter(1)
  issue(124, 1)
  wait_issue(123, 0)
  scale_chunk(123, 0)
  start_scatter(0)
  wait_issue(124, 1)
  scale_chunk(124, 1)
  start_scatter(1)

  for b in range(NBA):
    wait_scatter(b)

  plsc.subcore_barrier()

  @pl.when(sid < 10)
  def _out():
    for k in range(12):
      sl = pl.ds(sid * 1000 + k * CHA, CHA)
      pltpu.sync_copy(acc_sh.at[sl], rin[0])
      pltpu.sync_copy(rin[0], out_hbm.at[cid, sl])
    sl = pl.ds(sid * 1000 + 960, 40)
    pltpu.sync_copy(acc_sh.at[sl], rin[0].at[pl.ds(0, 40)])
    pltpu.sync_copy(rin[0].at[pl.ds(0, 40)], out_hbm.at[cid, sl])


_sc_agg = pl.kernel(
    _agg_body,
    out_type=jax.ShapeDtypeStruct((NC, N, H), jnp.float32),
    mesh=_mesh,
    compiler_params=pltpu.CompilerParams(needs_layout_passes=False),
    scratch_types=(
        [pltpu.VMEM((EPW,), jnp.float32)]
        + [pltpu.VMEM((CHA, H), jnp.float32)] * 3
        + [pltpu.VMEM((CHA,), jnp.int32)] * 6
        + [pltpu.SemaphoreType.DMA] * 9
        + [pltpu.VMEM_SHARED((N, H), jnp.float32)]
    ),
)


BR = 1000


def _tc1_body(degp_ref, x_ref, w1_ref, dinv_ref, g1_ref):
  deg = degp_ref[0] + degp_ref[1] + 1.0
  dinv = lax.rsqrt(deg)
  h = jnp.dot(x_ref[...], w1_ref[...], preferred_element_type=jnp.float32)
  dinv_ref[...] = dinv
  g1_ref[...] = dinv * h


def _tc_mid_body(acc_ref, g_ref, dinv_ref, b_ref, w_ref, gnext_ref):
  a = acc_ref[0] + acc_ref[1] + g_ref[...]
  o = jnp.maximum(dinv_ref[...] * a + b_ref[...], 0.0)
  gnext_ref[...] = dinv_ref[...] * jnp.dot(
      o, w_ref[...], preferred_element_type=jnp.float32)


def _tc_out_body(acc_ref, g_ref, dinv_ref, b_ref, w_ref, bfc_ref, y_ref):
  a = acc_ref[0] + acc_ref[1] + g_ref[...]
  o = jnp.maximum(dinv_ref[...] * a + b_ref[...], 0.0)
  y_ref[...] = jnp.dot(
      o, w_ref[...], preferred_element_type=jnp.float32) + bfc_ref[...]


_tc1 = pl.pallas_call(
    _tc1_body,
    grid=(N // BR,),
    in_specs=[
        pl.BlockSpec((NC, BR, H), lambda i: (0, i, 0)),
        pl.BlockSpec((BR, D), lambda i: (i, 0)),
        pl.BlockSpec((D, H), lambda i: (0, 0)),
    ],
    out_specs=[
        pl.BlockSpec((BR, H), lambda i: (i, 0)),
        pl.BlockSpec((BR, H), lambda i: (i, 0)),
    ],
    out_shape=[
        jax.ShapeDtypeStruct((N, H), jnp.float32),
        jax.ShapeDtypeStruct((N, H), jnp.float32),
    ],
)

_tc_mid = pl.pallas_call(
    _tc_mid_body,
    grid=(N // BR,),
    in_specs=[
        pl.BlockSpec((NC, BR, H), lambda i: (0, i, 0)),
        pl.BlockSpec((BR, H), lambda i: (i, 0)),
        pl.BlockSpec((BR, H), lambda i: (i, 0)),
        pl.BlockSpec((1, H), lambda i: (0, 0)),
        pl.BlockSpec((H, H), lambda i: (0, 0)),
    ],
    out_specs=pl.BlockSpec((BR, H), lambda i: (i, 0)),
    out_shape=jax.ShapeDtypeStruct((N, H), jnp.float32),
)

_tc_out = pl.pallas_call(
    _tc_out_body,
    grid=(N // BR,),
    in_specs=[
        pl.BlockSpec((NC, BR, H), lambda i: (0, i, 0)),
        pl.BlockSpec((BR, H), lambda i: (i, 0)),
        pl.BlockSpec((BR, H), lambda i: (i, 0)),
        pl.BlockSpec((1, H), lambda i: (0, 0)),
        pl.BlockSpec((H, OUT), lambda i: (0, 0)),
        pl.BlockSpec((1, OUT), lambda i: (0, 0)),
    ],
    out_specs=pl.BlockSpec((BR, OUT), lambda i: (i, 0)),
    out_shape=jax.ShapeDtypeStruct((N, OUT), jnp.float32),
)


def kernel(x, edge_index, edge_weight, W1, b1, W2, b2, Wfc, bfc):
  src = edge_index[0]
  dst = edge_index[1]
  w = edge_weight

  degp = _sc_deg(dst, w)
  degb = jnp.broadcast_to(degp.reshape(NC, N, 1), (NC, N, H))
  dinv, g1 = _tc1(degb, x, W1)
  acc1 = _sc_agg(g1, src, dst, w)
  g2 = _tc_mid(acc1, g1, dinv, b1.reshape(1, H), W2)
  acc2 = _sc_agg(g2, src, dst, w)
  return _tc_out(acc2, g2, dinv, b2.reshape(1, H), Wfc, bfc.reshape(1, OUT))


# speedup vs baseline: 25.9607x; 1.0012x over previous
import jax
import jax.numpy as jnp
from jax import lax
from jax.experimental import pallas as pl
from jax.experimental.pallas import tpu as pltpu
from jax.experimental.pallas import tpu_sc as plsc

N = 10000
E = 320000
D = 128
H = 128
OUT = 128

NC = 2
NS = 16
NW = NC * NS
EPW = E // NW
CH = 40
NCHUNK = EPW // CH
NB = 5
NGG = NCHUNK // (2 * NB)
CHA = 80
NCHA = EPW // CHA
NBA = 4

_mesh = plsc.VectorSubcoreMesh(
    core_axis_name="c", subcore_axis_name="s", num_cores=NC, num_subcores=NS)


def _deg_body(dst_hbm, w_hbm, out_hbm, w_all, zb_v,
              d00, d01, d02, d03, d04, d10, d11, d12, d13, d14,
              ds0, ds1, ds2, ds3, ds4, ss0, ss1, ss2, ss3, ss4, deg_sh):
  cid = lax.axis_index("c")
  sid = lax.axis_index("s")
  wid = cid * NS + sid
  dstb = ((d00, d01, d02, d03, d04), (d10, d11, d12, d13, d14))
  dsem = (ds0, ds1, ds2, ds3, ds4)
  ssem = (ss0, ss1, ss2, ss3, ss4)

  @pl.when(sid == 0)
  def _init():
    def z(i, carry):
      zb_v[pl.ds(i * 16, 16)] = jnp.zeros((16,), jnp.float32)
      return carry
    lax.fori_loop(0, N // 16, z, None)
    pltpu.sync_copy(zb_v, deg_sh)

  pltpu.sync_copy(w_hbm.at[pl.ds(wid * EPW, EPW)], w_all)
  plsc.subcore_barrier()

  for b in range(NB):
    pltpu.async_copy(dst_hbm.at[pl.ds(wid * EPW + b * CH, CH)],
                     dstb[0][b], dsem[b])

  def rounds(gg, carry):
    for p in range(2):
      for b in range(NB):
        c = (2 * gg + p) * NB + b
        base = c * CH
        pltpu.make_async_copy(
            dst_hbm.at[pl.ds(wid * EPW + base, CH)], dstb[p][b],
            dsem[b]).wait()
        def wait_prev():
          pltpu.make_async_copy(
              w_all.at[pl.ds(base, CH)], deg_sh.at[dstb[1 - p][b]],
              ssem[b]).wait()
        if p == 1:
          wait_prev()
        else:
          pl.when(gg > 0)(wait_prev)
        pltpu.async_copy(w_all.at[pl.ds(base, CH)], deg_sh.at[dstb[p][b]],
                         ssem[b], add=True)
        def prefetch():
          pltpu.async_copy(
              dst_hbm.at[pl.ds(wid * EPW + base + NB * CH, CH)],
              dstb[1 - p][b], dsem[b])
        if p == 0:
          prefetch()
        else:
          pl.when(gg < NGG - 1)(prefetch)
    return carry
  lax.fori_loop(0, NGG, rounds, None)

  for b in range(NB):
    pltpu.make_async_copy(w_all.at[pl.ds(0, CH)], deg_sh.at[dstb[1][b]],
                          ssem[b]).wait()

  plsc.subcore_barrier()

  @pl.when(sid < 10)
  def _out():
    stage = zb_v.at[pl.ds(0, 1000)]
    pltpu.sync_copy(deg_sh.at[pl.ds(sid * 1000, 1000)], stage)
    pltpu.sync_copy(stage, out_hbm.at[pl.ds(cid * N + sid * 1000, 1000)])


_sc_deg = pl.kernel(
    _deg_body,
    out_type=jax.ShapeDtypeStruct((NC * N,), jnp.float32),
    mesh=_mesh,
    compiler_params=pltpu.CompilerParams(needs_layout_passes=False),
    scratch_types=(
        [pltpu.VMEM((EPW,), jnp.float32),
         pltpu.VMEM((N,), jnp.float32)]
        + [pltpu.VMEM((CH,), jnp.int32)] * 10
        + [pltpu.SemaphoreType.DMA] * 10
        + [pltpu.VMEM_SHARED((N,), jnp.float32)]
    ),
)


def _agg_body(g_hbm, src_hbm, dst_hbm, w_hbm, out_hbm,
              ri0, ri1, ri2, ri3, d0, d1, d2, d3, s0, s1, s2, s3,
              wb0, wb1, wb2, wb3,
              gs0, gs1, gs2, gs3, ss0, ss1, ss2, ss3, ps0, ps1, ps2, ps3,
              acc_sh):
  cid = lax.axis_index("c")
  sid = lax.axis_index("s")
  wid = cid * NS + sid
  rin = (ri0, ri1, ri2, ri3)
  dstb = (d0, d1, d2, d3)
  srcb = (s0, s1, s2, s3)
  wb = (wb0, wb1, wb2, wb3)
  gsem = (gs0, gs1, gs2, gs3)
  ssem = (ss0, ss1, ss2, ss3)
  psem = (ps0, ps1, ps2, ps3)
  NR = 31

  def prefetch_sw(c, b):
    pltpu.async_copy(src_hbm.at[pl.ds(wid * EPW + c * CHA, CHA)],
                     srcb[b], psem[b])
    pltpu.async_copy(w_hbm.at[pl.ds(wid * EPW + c * CHA, CHA)],
                     wb[b], psem[b])

  def issue(c, b):
    pltpu.make_async_copy(src_hbm.at[pl.ds(wid * EPW + c * CHA, CHA)],
                          srcb[b], psem[b]).wait()
    pltpu.make_async_copy(w_hbm.at[pl.ds(wid * EPW + c * CHA, CHA)],
                          wb[b], psem[b]).wait()
    pltpu.async_copy(dst_hbm.at[pl.ds(wid * EPW + c * CHA, CHA)],
                     dstb[b], gsem[b])
    pltpu.async_copy(g_hbm.at[srcb[b]], rin[b], gsem[b])

  def wait_issue(c, b):
    pltpu.make_async_copy(dst_hbm.at[pl.ds(wid * EPW + c * CHA, CHA)],
                          dstb[b], gsem[b]).wait()
    pltpu.make_async_copy(g_hbm.at[srcb[b]], rin[b], gsem[b]).wait()

  def wait_scatter(b):
    pltpu.make_async_copy(rin[b], acc_sh.at[dstb[b]], ssem[b]).wait()

  def scale_chunk(c, b):
    def scale(i, c2):
      for u in range(2):
        e = 2 * i + u
        wsplat = plsc.load_gather(wb[b], [jnp.full((16,), e, jnp.int32)])
        for j in range(H // 16):
          sl = pl.ds(j * 16, 16)
          rin[b][e, sl] = rin[b][e, sl] * wsplat
      return c2
    lax.fori_loop(0, CHA // 2, scale, None)

  def start_scatter(b):
    pltpu.async_copy(rin[b], acc_sh.at[dstb[b]], ssem[b], add=True)

  def z(i, carry):
    for j in range(H // 16):
      rin[0][i, pl.ds(j * 16, 16)] = jnp.zeros((16,), jnp.float32)
    return carry
  lax.fori_loop(0, CHA, z, None)

  @pl.when(sid < 10)
  def _zero():
    for k in range(12):
      pltpu.sync_copy(rin[0], acc_sh.at[pl.ds(sid * 1000 + k * CHA, CHA)])
    pltpu.sync_copy(rin[0].at[pl.ds(0, 40)],
                    acc_sh.at[pl.ds(sid * 1000 + 960, 40)])

  plsc.subcore_barrier()

  prefetch_sw(0, 0)
  prefetch_sw(1, 1)
  issue(0, 0)

  def rounds(r, carry):
    for b in range(NBA):
      c = r * NBA + b
      bn = (b + 1) % NBA
      b2 = (b + 2) % NBA
      def advance():
        pl.when(r > 0)(lambda: wait_scatter(bn)) if b < NBA - 1 \
            else wait_scatter(bn)
        issue(c + 1, bn)
      advance()
      if b < NBA - 1:
        prefetch_sw(c + 2, b2)
      else:
        pl.when(r < NR - 1)(lambda: prefetch_sw(c + 2, b2))
      wait_issue(c, b)
      scale_chunk(c, b)
      start_scatter(b)
    return carry
  lax.fori_loop(0, NR, rounds, None)

  wait_issue(124, 0)
  scale_chunk(124, 0)
  start_scatter(0)

  for b in range(NBA):
    wait_scatter(b)

  plsc.subcore_barrier()

  @pl.when(sid < 10)
  def _out():
    for k in range(12):
      sl = pl.ds(sid * 1000 + k * CHA, CHA)
      pltpu.sync_copy(acc_sh.at[sl], rin[0])
      pltpu.sync_copy(rin[0], out_hbm.at[cid, sl])
    sl = pl.ds(sid * 1000 + 960, 40)
    pltpu.sync_copy(acc_sh.at[sl], rin[0].at[pl.ds(0, 40)])
    pltpu.sync_copy(rin[0].at[pl.ds(0, 40)], out_hbm.at[cid, sl])


_sc_agg = pl.kernel(
    _agg_body,
    out_type=jax.ShapeDtypeStruct((NC, N, H), jnp.float32),
    mesh=_mesh,
    compiler_params=pltpu.CompilerParams(needs_layout_passes=False),
    scratch_types=(
        [pltpu.VMEM((CHA, H), jnp.float32)] * 4
        + [pltpu.VMEM((CHA,), jnp.int32)] * 8
        + [pltpu.VMEM((CHA,), jnp.float32)] * 4
        + [pltpu.SemaphoreType.DMA] * 12
        + [pltpu.VMEM_SHARED((N, H), jnp.float32)]
    ),
)


BR = 1000


def _tc1_body(degp_ref, x_ref, w1_ref, dinv_ref, g1_ref):
  deg = degp_ref[0] + degp_ref[1] + 1.0
  dinv = lax.rsqrt(deg)
  h = jnp.dot(x_ref[...], w1_ref[...], preferred_element_type=jnp.float32)
  dinv_ref[...] = dinv
  g1_ref[...] = dinv * h


def _tc_mid_body(acc_ref, g_ref, dinv_ref, b_ref, w_ref, gnext_ref):
  a = acc_ref[0] + acc_ref[1] + g_ref[...]
  o = jnp.maximum(dinv_ref[...] * a + b_ref[...], 0.0)
  gnext_ref[...] = dinv_ref[...] * jnp.dot(
      o, w_ref[...], preferred_element_type=jnp.float32)


def _tc_out_body(acc_ref, g_ref, dinv_ref, b_ref, w_ref, bfc_ref, y_ref):
  a = acc_ref[0] + acc_ref[1] + g_ref[...]
  o = jnp.maximum(dinv_ref[...] * a + b_ref[...], 0.0)
  y_ref[...] = jnp.dot(
      o, w_ref[...], preferred_element_type=jnp.float32) + bfc_ref[...]


_tc1 = pl.pallas_call(
    _tc1_body,
    grid=(N // BR,),
    in_specs=[
        pl.BlockSpec((NC, BR, H), lambda i: (0, i, 0)),
        pl.BlockSpec((BR, D), lambda i: (i, 0)),
        pl.BlockSpec((D, H), lambda i: (0, 0)),
    ],
    out_specs=[
        pl.BlockSpec((BR, H), lambda i: (i, 0)),
        pl.BlockSpec((BR, H), lambda i: (i, 0)),
    ],
    out_shape=[
        jax.ShapeDtypeStruct((N, H), jnp.float32),
        jax.ShapeDtypeStruct((N, H), jnp.float32),
    ],
)

_tc_mid = pl.pallas_call(
    _tc_mid_body,
    grid=(N // BR,),
    in_specs=[
        pl.BlockSpec((NC, BR, H), lambda i: (0, i, 0)),
        pl.BlockSpec((BR, H), lambda i: (i, 0)),
        pl.BlockSpec((BR, H), lambda i: (i, 0)),
        pl.BlockSpec((1, H), lambda i: (0, 0)),
        pl.BlockSpec((H, H), lambda i: (0, 0)),
    ],
    out_specs=pl.BlockSpec((BR, H), lambda i: (i, 0)),
    out_shape=jax.ShapeDtypeStruct((N, H), jnp.float32),
)

_tc_out = pl.pallas_call(
    _tc_out_body,
    grid=(N // BR,),
    in_specs=[
        pl.BlockSpec((NC, BR, H), lambda i: (0, i, 0)),
        pl.BlockSpec((BR, H), lambda i: (i, 0)),
        pl.BlockSpec((BR, H), lambda i: (i, 0)),
        pl.BlockSpec((1, H), lambda i: (0, 0)),
        pl.BlockSpec((H, OUT), lambda i: (0, 0)),
        pl.BlockSpec((1, OUT), lambda i: (0, 0)),
    ],
    out_specs=pl.BlockSpec((BR, OUT), lambda i: (i, 0)),
    out_shape=jax.ShapeDtypeStruct((N, OUT), jnp.float32),
)


def kernel(x, edge_index, edge_weight, W1, b1, W2, b2, Wfc, bfc):
  src = edge_index[0]
  dst = edge_index[1]
  w = edge_weight

  degp = _sc_deg(dst, w)
  degb = jnp.broadcast_to(degp.reshape(NC, N, 1), (NC, N, H))
  dinv, g1 = _tc1(degb, x, W1)
  acc1 = _sc_agg(g1, src, dst, w)
  g2 = _tc_mid(acc1, g1, dinv, b1.reshape(1, H), W2)
  acc2 = _sc_agg(g2, src, dst, w)
  return _tc_out(acc2, g2, dinv, b2.reshape(1, H), Wfc, bfc.reshape(1, OUT))


# speedup vs baseline: 25.9871x; 1.0010x over previous
import jax
import jax.numpy as jnp
from jax import lax
from jax.experimental import pallas as pl
from jax.experimental.pallas import tpu as pltpu
from jax.experimental.pallas import tpu_sc as plsc

N = 10000
E = 320000
D = 128
H = 128
OUT = 128

NC = 2
NS = 16
NW = NC * NS
EPW = E // NW
CH = 40
NCHUNK = EPW // CH
NB = 5
NGG = NCHUNK // (2 * NB)
CHA = 80
NCHA = EPW // CHA
NBA = 4

_mesh = plsc.VectorSubcoreMesh(
    core_axis_name="c", subcore_axis_name="s", num_cores=NC, num_subcores=NS)


def _deg_body(dst_hbm, w_hbm, out_hbm, w_all, zb_v,
              d00, d01, d02, d03, d04, d10, d11, d12, d13, d14,
              ds0, ds1, ds2, ds3, ds4, ss0, ss1, ss2, ss3, ss4, deg_sh):
  cid = lax.axis_index("c")
  sid = lax.axis_index("s")
  wid = cid * NS + sid
  dstb = ((d00, d01, d02, d03, d04), (d10, d11, d12, d13, d14))
  dsem = (ds0, ds1, ds2, ds3, ds4)
  ssem = (ss0, ss1, ss2, ss3, ss4)

  @pl.when(sid == 0)
  def _init():
    def z(i, carry):
      zb_v[pl.ds(i * 16, 16)] = jnp.zeros((16,), jnp.float32)
      return carry
    lax.fori_loop(0, N // 16, z, None)
    pltpu.sync_copy(zb_v, deg_sh)

  pltpu.sync_copy(w_hbm.at[pl.ds(wid * EPW, EPW)], w_all)
  plsc.subcore_barrier()

  for b in range(NB):
    pltpu.async_copy(dst_hbm.at[pl.ds(wid * EPW + b * CH, CH)],
                     dstb[0][b], dsem[b])

  def rounds(gg, carry):
    for p in range(2):
      for b in range(NB):
        c = (2 * gg + p) * NB + b
        base = c * CH
        pltpu.make_async_copy(
            dst_hbm.at[pl.ds(wid * EPW + base, CH)], dstb[p][b],
            dsem[b]).wait()
        def wait_prev():
          pltpu.make_async_copy(
              w_all.at[pl.ds(base, CH)], deg_sh.at[dstb[1 - p][b]],
              ssem[b]).wait()
        if p == 1:
          wait_prev()
        else:
          pl.when(gg > 0)(wait_prev)
        pltpu.async_copy(w_all.at[pl.ds(base, CH)], deg_sh.at[dstb[p][b]],
                         ssem[b], add=True)
        def prefetch():
          pltpu.async_copy(
              dst_hbm.at[pl.ds(wid * EPW + base + NB * CH, CH)],
              dstb[1 - p][b], dsem[b])
        if p == 0:
          prefetch()
        else:
          pl.when(gg < NGG - 1)(prefetch)
    return carry
  lax.fori_loop(0, NGG, rounds, None)

  for b in range(NB):
    pltpu.make_async_copy(w_all.at[pl.ds(0, CH)], deg_sh.at[dstb[1][b]],
                          ssem[b]).wait()

  plsc.subcore_barrier()

  @pl.when(sid < 10)
  def _out():
    stage = zb_v.at[pl.ds(0, 1000)]
    pltpu.sync_copy(deg_sh.at[pl.ds(sid * 1000, 1000)], stage)
    pltpu.sync_copy(stage, out_hbm.at[pl.ds(cid * N + sid * 1000, 1000)])


_sc_deg = pl.kernel(
    _deg_body,
    out_type=jax.ShapeDtypeStruct((NC * N,), jnp.float32),
    mesh=_mesh,
    compiler_params=pltpu.CompilerParams(needs_layout_passes=False),
    scratch_types=(
        [pltpu.VMEM((EPW,), jnp.float32),
         pltpu.VMEM((N,), jnp.float32)]
        + [pltpu.VMEM((CH,), jnp.int32)] * 10
        + [pltpu.SemaphoreType.DMA] * 10
        + [pltpu.VMEM_SHARED((N,), jnp.float32)]
    ),
)


def _agg_body(g_hbm, src_hbm, dst_hbm, w_hbm, out_hbm,
              ri0, ri1, ri2, ri3, d0, d1, d2, d3, s0, s1, s2, s3,
              wb0, wb1, wb2, wb3,
              gs0, gs1, gs2, gs3, ss0, ss1, ss2, ss3, ps0, ps1, ps2, ps3,
              acc_sh):
  cid = lax.axis_index("c")
  sid = lax.axis_index("s")
  wid = cid * NS + sid
  rin = (ri0, ri1, ri2, ri3)
  rout = rin
  dstb = (d0, d1, d2, d3)
  srcb = (s0, s1, s2, s3)
  wb = (wb0, wb1, wb2, wb3)
  gsem = (gs0, gs1, gs2, gs3)
  ssem = (ss0, ss1, ss2, ss3)
  psem = (ps0, ps1, ps2, ps3)
  NR = 31

  def prefetch_sw(c, b):
    pltpu.async_copy(src_hbm.at[pl.ds(wid * EPW + c * CHA, CHA)],
                     srcb[b], psem[b])
    pltpu.async_copy(w_hbm.at[pl.ds(wid * EPW + c * CHA, CHA)],
                     wb[b], psem[b])

  def issue(c, b):
    pltpu.make_async_copy(src_hbm.at[pl.ds(wid * EPW + c * CHA, CHA)],
                          srcb[b], psem[b]).wait()
    pltpu.make_async_copy(w_hbm.at[pl.ds(wid * EPW + c * CHA, CHA)],
                          wb[b], psem[b]).wait()
    pltpu.async_copy(dst_hbm.at[pl.ds(wid * EPW + c * CHA, CHA)],
                     dstb[b], gsem[b])
    pltpu.async_copy(g_hbm.at[srcb[b]], rin[b], gsem[b])

  def wait_issue(c, b):
    pltpu.make_async_copy(dst_hbm.at[pl.ds(wid * EPW + c * CHA, CHA)],
                          dstb[b], gsem[b]).wait()
    pltpu.make_async_copy(g_hbm.at[srcb[b]], rin[b], gsem[b]).wait()

  def wait_scatter(b):
    pltpu.make_async_copy(rout[b], acc_sh.at[dstb[b]], ssem[b]).wait()

  def scale_chunk(c, b):
    def scale(i, c2):
      for u in range(2):
        e = 2 * i + u
        wsplat = plsc.load_gather(wb[b], [jnp.full((16,), e, jnp.int32)])
        for j in range(H // 16):
          sl = pl.ds(j * 16, 16)
          rin[b][e, sl] = rin[b][e, sl] * wsplat
      return c2
    lax.fori_loop(0, CHA // 2, scale, None)

  def start_scatter(b):
    pltpu.async_copy(rout[b], acc_sh.at[dstb[b]], ssem[b], add=True)

  def z(i, carry):
    for j in range(H // 16):
      rout[0][i, pl.ds(j * 16, 16)] = jnp.zeros((16,), jnp.float32)
    return carry
  lax.fori_loop(0, CHA, z, None)

  @pl.when(sid < 10)
  def _zero():
    for k in range(12):
      pltpu.sync_copy(rout[0], acc_sh.at[pl.ds(sid * 1000 + k * CHA, CHA)])
    pltpu.sync_copy(rout[0].at[pl.ds(0, 40)],
                    acc_sh.at[pl.ds(sid * 1000 + 960, 40)])

  plsc.subcore_barrier()

  prefetch_sw(0, 0)
  prefetch_sw(1, 1)
  issue(0, 0)

  def rounds(r, carry):
    for b in range(NBA):
      c = r * NBA + b
      bn = (b + 1) % NBA
      b2 = (b + 2) % NBA
      def advance():
        pl.when(r > 0)(lambda: wait_scatter(bn)) if b < NBA - 1 \
            else wait_scatter(bn)
        issue(c + 1, bn)
      advance()
      if b < NBA - 1:
        prefetch_sw(c + 2, b2)
      else:
        pl.when(r < NR - 1)(lambda: prefetch_sw(c + 2, b2))
      wait_issue(c, b)
      scale_chunk(c, b)
      start_scatter(b)
    return carry
  lax.fori_loop(0, NR, rounds, None)

  wait_issue(124, 0)
  scale_chunk(124, 0)
  start_scatter(0)

  for b in range(NBA):
    wait_scatter(b)

  plsc.subcore_barrier()

  @pl.when(sid < 10)
  def _out():
    for k in range(12):
      sl = pl.ds(sid * 1000 + k * CHA, CHA)
      pltpu.sync_copy(acc_sh.at[sl], rout[0])
      pltpu.sync_copy(rout[0], out_hbm.at[cid, sl])
    sl = pl.ds(sid * 1000 + 960, 40)
    pltpu.sync_copy(acc_sh.at[sl], rout[0].at[pl.ds(0, 40)])
    pltpu.sync_copy(rout[0].at[pl.ds(0, 40)], out_hbm.at[cid, sl])


_sc_agg = pl.kernel(
    _agg_body,
    out_type=jax.ShapeDtypeStruct((NC, N, H), jnp.float32),
    mesh=_mesh,
    compiler_params=pltpu.CompilerParams(needs_layout_passes=False),
    scratch_types=(
        [pltpu.VMEM((CHA, H), jnp.float32)] * 4
        + [pltpu.VMEM((CHA,), jnp.int32)] * 8
        + [pltpu.VMEM((CHA,), jnp.float32)] * 4
        + [pltpu.SemaphoreType.DMA] * 12
        + [pltpu.VMEM_SHARED((N, H), jnp.float32)]
    ),
)


BR = 1000


def _tc1_body(degp_ref, x_ref, w1_ref, dinv_ref, g1_ref):
  deg = degp_ref[0] + degp_ref[1] + 1.0
  dinv = lax.rsqrt(deg)
  h = jnp.dot(x_ref[...], w1_ref[...], preferred_element_type=jnp.float32)
  dinv_ref[...] = dinv
  g1_ref[...] = dinv * h


def _tc_mid_body(acc_ref, g_ref, dinv_ref, b_ref, w_ref, gnext_ref):
  a = acc_ref[0] + acc_ref[1] + g_ref[...]
  o = jnp.maximum(dinv_ref[...] * a + b_ref[...], 0.0)
  gnext_ref[...] = dinv_ref[...] * jnp.dot(
      o, w_ref[...], preferred_element_type=jnp.float32)


def _tc_out_body(acc_ref, g_ref, dinv_ref, b_ref, w_ref, bfc_ref, y_ref):
  a = acc_ref[0] + acc_ref[1] + g_ref[...]
  o = jnp.maximum(dinv_ref[...] * a + b_ref[...], 0.0)
  y_ref[...] = jnp.dot(
      o, w_ref[...], preferred_element_type=jnp.float32) + bfc_ref[...]


_tc1 = pl.pallas_call(
    _tc1_body,
    grid=(N // BR,),
    in_specs=[
        pl.BlockSpec((NC, BR, H), lambda i: (0, i, 0)),
        pl.BlockSpec((BR, D), lambda i: (i, 0)),
        pl.BlockSpec((D, H), lambda i: (0, 0)),
    ],
    out_specs=[
        pl.BlockSpec((BR, H), lambda i: (i, 0)),
        pl.BlockSpec((BR, H), lambda i: (i, 0)),
    ],
    out_shape=[
        jax.ShapeDtypeStruct((N, H), jnp.float32),
        jax.ShapeDtypeStruct((N, H), jnp.float32),
    ],
)

_tc_mid = pl.pallas_call(
    _tc_mid_body,
    grid=(N // BR,),
    in_specs=[
        pl.BlockSpec((NC, BR, H), lambda i: (0, i, 0)),
        pl.BlockSpec((BR, H), lambda i: (i, 0)),
        pl.BlockSpec((BR, H), lambda i: (i, 0)),
        pl.BlockSpec((1, H), lambda i: (0, 0)),
        pl.BlockSpec((H, H), lambda i: (0, 0)),
    ],
    out_specs=pl.BlockSpec((BR, H), lambda i: (i, 0)),
    out_shape=jax.ShapeDtypeStruct((N, H), jnp.float32),
)

_tc_out = pl.pallas_call(
    _tc_out_body,
    grid=(N // BR,),
    in_specs=[
        pl.BlockSpec((NC, BR, H), lambda i: (0, i, 0)),
        pl.BlockSpec((BR, H), lambda i: (i, 0)),
        pl.BlockSpec((BR, H), lambda i: (i, 0)),
        pl.BlockSpec((1, H), lambda i: (0, 0)),
        pl.BlockSpec((H, OUT), lambda i: (0, 0)),
        pl.BlockSpec((1, OUT), lambda i: (0, 0)),
    ],
    out_specs=pl.BlockSpec((BR, OUT), lambda i: (i, 0)),
    out_shape=jax.ShapeDtypeStruct((N, OUT), jnp.float32),
)


def kernel(x, edge_index, edge_weight, W1, b1, W2, b2, Wfc, bfc):
  src = edge_index[0]
  dst = edge_index[1]
  w = edge_weight

  degp = _sc_deg(dst, w)
  degb = jnp.broadcast_to(degp.reshape(NC, N, 1), (NC, N, H))
  dinv, g1 = _tc1(degb, x, W1)
  acc1 = _sc_agg(g1, src, dst, w)
  g2 = _tc_mid(acc1, g1, dinv, b1.reshape(1, H), W2)
  acc2 = _sc_agg(g2, src, dst, w)
  return _tc_out(acc2, g2, dinv, b2.reshape(1, H), Wfc, bfc.reshape(1, OUT))
